# Initial kernel scaffold; baseline (speedup 1.0000x reference)
import jax
import jax.numpy as jnp
from jax.experimental import pallas as pl


def kernel(x, edge_index, bn_gamma, bn_beta, W, b):
    raise NotImplementedError("write your pallas kernel here")


# speedup vs baseline: 14.7257x; 14.7257x over previous
import functools

import jax
import jax.numpy as jnp
from jax import lax
from jax.experimental import pallas as pl
from jax.experimental.pallas import tpu as pltpu
from jax.experimental.pallas import tpu_sc as plsc

N = 10000
C = 128
E = 320000
NC = 2
NS = 16
NW = NC * NS
EPT = E // NW
CHUNK = 80
NCHUNK = EPT // CHUNK
NPAD = 10112
ROWS = NPAD // NS
DW = 128

_mesh = plsc.VectorSubcoreMesh(
    core_axis_name="c", subcore_axis_name="s", num_cores=NC, num_subcores=NS)


@functools.partial(
    pl.kernel,
    out_type=jax.ShapeDtypeStruct((NC, NPAD, DW), jnp.float32),
    mesh=_mesh,
    scratch_types=[
        pltpu.VMEM((CHUNK,), jnp.int32),
        pltpu.VMEM((CHUNK, DW), jnp.float32),
        pltpu.VMEM_SHARED((NPAD, DW), jnp.float32),
    ],
)
def _deg_kernel(dst_hbm, ones_hbm, zeros_hbm, out_hbm, idx_v, ones_v, acc):
    cid = lax.axis_index("c")
    sid = lax.axis_index("s")
    tid = sid * NC + cid
    base = tid * EPT
    pltpu.sync_copy(ones_hbm, ones_v)
    pltpu.sync_copy(zeros_hbm, acc.at[pl.ds(sid * ROWS, ROWS)])
    plsc.subcore_barrier()

    def body(i, carry):
        pltpu.sync_copy(dst_hbm.at[pl.ds(base + i * CHUNK, CHUNK)], idx_v)
        pltpu.sync_copy(ones_v, acc.at[idx_v], add=True)
        return carry

    lax.fori_loop(0, NCHUNK, body, 0)
    plsc.subcore_barrier()
    pltpu.sync_copy(acc.at[pl.ds(sid * ROWS, ROWS)],
                    out_hbm.at[cid, pl.ds(sid * ROWS, ROWS)])


@functools.partial(
    pl.kernel,
    out_type=jax.ShapeDtypeStruct((NC, NPAD, C), jnp.float32),
    mesh=_mesh,
    scratch_types=[
        pltpu.VMEM((CHUNK,), jnp.int32),
        pltpu.VMEM((CHUNK,), jnp.int32),
        pltpu.VMEM((CHUNK, C), jnp.float32),
        pltpu.SemaphoreType.DMA,
        pltpu.VMEM_SHARED((NPAD, C), jnp.float32),
    ],
)
def _scatter_kernel(src_hbm, dst_hbm, y_hbm, zeros_hbm, out_hbm,
                    si_v, di_v, rows_v, sem, acc):
    cid = lax.axis_index("c")
    sid = lax.axis_index("s")
    tid = sid * NC + cid
    base = tid * EPT
    pltpu.sync_copy(zeros_hbm, acc.at[pl.ds(sid * ROWS, ROWS)])
    plsc.subcore_barrier()

    def body(i, carry):
        e0 = base + i * CHUNK
        pltpu.sync_copy(src_hbm.at[pl.ds(e0, CHUNK)], si_v)
        pltpu.sync_copy(dst_hbm.at[pl.ds(e0, CHUNK)], di_v)
        pltpu.async_copy(y_hbm.at[si_v], rows_v, sem).wait()
        pltpu.sync_copy(rows_v, acc.at[di_v], add=True)
        return carry

    lax.fori_loop(0, NCHUNK, body, 0)
    plsc.subcore_barrier()
    pltpu.sync_copy(acc.at[pl.ds(sid * ROWS, ROWS)],
                    out_hbm.at[cid, pl.ds(sid * ROWS, ROWS)])


def _bnmm_body(x_ref, g_ref, be_ref, w_ref, degp_ref, y_ref):
    x = x_ref[...]
    mean = jnp.mean(x, axis=0, keepdims=True)
    xc = x - mean
    var = jnp.mean(xc * xc, axis=0, keepdims=True)
    xh = xc * lax.rsqrt(var + 1e-5) * g_ref[...] + be_ref[...]
    xw = jnp.dot(xh, w_ref[...], preferred_element_type=jnp.float32)
    deg = degp_ref[0, 0:N, 0:1] + degp_ref[1, 0:N, 0:1] + 1.0
    y_ref[...] = xw * lax.rsqrt(deg)


_bnmm_call = pl.pallas_call(
    _bnmm_body, out_shape=jax.ShapeDtypeStruct((N, C), jnp.float32))


def _fin_body(p_ref, y_ref, degp_ref, b_ref, o_ref):
    deg = degp_ref[0, 0:N, 0:1] + degp_ref[1, 0:N, 0:1] + 1.0
    dis = lax.rsqrt(deg)
    s = p_ref[0, 0:N] + p_ref[1, 0:N] + y_ref[...]
    o_ref[...] = jnp.maximum(s * dis + b_ref[...], 0.0)


_fin_call = pl.pallas_call(
    _fin_body, out_shape=jax.ShapeDtypeStruct((N, C), jnp.float32))


def kernel(x, edge_index, bn_gamma, bn_beta, W, b):
    src = edge_index[0].astype(jnp.int32)
    dst = edge_index[1].astype(jnp.int32)
    ones_deg = jnp.ones((CHUNK, DW), jnp.float32)
    zeros_deg = jnp.zeros((ROWS, DW), jnp.float32)
    zeros_c = jnp.zeros((ROWS, C), jnp.float32)
    degp = _deg_kernel(dst, ones_deg, zeros_deg)
    y = _bnmm_call(x, bn_gamma.reshape(1, C), bn_beta.reshape(1, C), W, degp)
    p = _scatter_kernel(src, dst, y, zeros_c)
    return _fin_call(p, y, degp, b.reshape(1, C))
